# baseline (speedup 1.0000x reference)
import functools

import jax
import jax.numpy as jnp
from jax import lax
from jax.experimental import pallas as pl
from jax.experimental.pallas import tpu as pltpu

_HIGH = lax.Precision.HIGHEST


def _moe_kernel(x_ref, wr_ref, br_ref, up0_ref, up1_ref, dn0_ref, dn1_ref,
                out_ref, mall_s, mw_s, xg_ref, nblk_s, *, T, E, CAP, SUB, HCH, NHC):
    e = pl.program_id(0)

    @pl.when(e == 0)
    def _():
        x = x_ref[...]
        logits = lax.dot_general(x, wr_ref[...], (((1,), (1,)), ((), ())),
                                 preferred_element_type=jnp.float32)
        logits = logits + br_ref[...]
        m = jnp.max(logits, axis=1, keepdims=True)
        p = jnp.exp(logits - m)
        p = p / jnp.sum(p, axis=1, keepdims=True)

        ie = lax.broadcasted_iota(jnp.int32, (T, E), 1)
        m1 = jnp.max(p, axis=1, keepdims=True)
        am1 = jnp.min(jnp.where(p == m1, ie, E), axis=1, keepdims=True)
        pm = jnp.where(ie == am1, -1.0, p)
        m2 = jnp.max(pm, axis=1, keepdims=True)
        am2 = jnp.min(jnp.where(pm == m2, ie, E), axis=1, keepdims=True)

        oh1 = (ie == am1).astype(jnp.float32)
        oh2 = (ie == am2).astype(jnp.float32)
        both = oh1 + oh2

        it = lax.broadcasted_iota(jnp.int32, (T, T), 0)
        jt = lax.broadcasted_iota(jnp.int32, (T, T), 1)
        Ltri = (jt < it).astype(jnp.float32)
        cnt = lax.dot_general(Ltri, both, (((1,), (0,)), ((), ())),
                              preferred_element_type=jnp.float32, precision=_HIGH)
        r0 = jnp.sum(oh1 * cnt, axis=1, keepdims=True)
        r1 = jnp.sum(oh2 * cnt, axis=1, keepdims=True)

        ne = jnp.sum(both, axis=0, keepdims=True)
        nblk = jnp.ceil(ne * (1.0 / SUB)).astype(jnp.int32)
        for ee in range(E):
            nblk_s[0, ee] = nblk[0, ee]

        f0 = am1 * CAP + (r0 + 0.5).astype(jnp.int32)
        f1 = am2 * CAP + (r1 + 0.5).astype(jnp.int32)
        fcap = lax.broadcasted_iota(jnp.int32, (T, CAP), 1)
        for ee in range(E):
            fi = fcap + ee * CAP
            M0e = (fi == f0).astype(jnp.float32)
            M1e = (fi == f1).astype(jnp.float32)
            mall_s[ee] = M0e + M1e
            mw_s[ee] = M0e * m1 + M1e * m2

        out_ref[...] = jnp.zeros_like(out_ref)

    nblk = nblk_s[0, e]
    xg_ref[...] = lax.dot_general(mall_s[e], x_ref[...],
                                  (((0,), (0,)), ((), ())),
                                  preferred_element_type=jnp.float32)

    for sub in range(CAP // SUB):
        @pl.when(sub < nblk)
        def _():
            xs = xg_ref[sub * SUB:(sub + 1) * SUB, :]
            contrib = None
            for c in range(NHC):
                up_c = (up0_ref, up1_ref)[c][0]
                dn_c = (dn0_ref, dn1_ref)[c][0]
                hp = lax.dot_general(xs, up_c, (((1,), (1,)), ((), ())),
                                     preferred_element_type=jnp.float32)
                g = hp * (0.5 * (1.0 + lax.erf(hp * 0.7071067811865476)))
                d = lax.dot_general(g, dn_c, (((1,), (1,)), ((), ())),
                                    preferred_element_type=jnp.float32)
                contrib = d if contrib is None else contrib + d
            mws = mw_s[e][:, sub * SUB:(sub + 1) * SUB]
            out_ref[...] += lax.dot_general(mws, contrib, (((1,), (0,)), ((), ())),
                                            preferred_element_type=jnp.float32)


def kernel(x, Wr, br, expert_up, expert_down):
    Bsz, Ssz, D = x.shape
    E, H = expert_up.shape[0], expert_up.shape[1]
    T = Bsz * Ssz
    CAP = T
    SUB = 128
    HCH = 1536
    NHC = H // HCH
    xf = x.reshape(T, D)

    body = functools.partial(_moe_kernel, T=T, E=E, CAP=CAP, SUB=SUB,
                             HCH=HCH, NHC=NHC)
    out = pl.pallas_call(
        body,
        grid=(E,),
        in_specs=[
            pl.BlockSpec((T, D), lambda e: (0, 0)),
            pl.BlockSpec((E, D), lambda e: (0, 0)),
            pl.BlockSpec((1, E), lambda e: (0, 0)),
            pl.BlockSpec((1, HCH, D), lambda e: (e, 0, 0)),
            pl.BlockSpec((1, HCH, D), lambda e: (e, 1, 0)),
            pl.BlockSpec((1, D, HCH), lambda e: (e, 0, 0)),
            pl.BlockSpec((1, D, HCH), lambda e: (e, 0, 1)),
        ],
        out_specs=pl.BlockSpec((T, D), lambda e: (0, 0)),
        scratch_shapes=[
            pltpu.VMEM((E, T, CAP), jnp.float32),
            pltpu.VMEM((E, T, CAP), jnp.float32),
            pltpu.VMEM((CAP, D), jnp.float32),
            pltpu.SMEM((1, E), jnp.int32),
        ],
        out_shape=jax.ShapeDtypeStruct((T, D), jnp.float32),
    )(xf, Wr, br.reshape(1, E), expert_up, expert_up, expert_down, expert_down)
    return out.reshape(Bsz, Ssz, D)


# speedup vs baseline: 1.0044x; 1.0044x over previous
import functools

import jax
import jax.numpy as jnp
from jax import lax
from jax.experimental import pallas as pl
from jax.experimental.pallas import tpu as pltpu

_HIGH = lax.Precision.HIGHEST


def _moe_kernel(x_ref, wr_ref, br_ref, up0_ref, up1_ref, dn0_ref, dn1_ref,
                out_ref, mall_s, mw_s, xg_ref, nblk_s, *, T, E, CAP, SUB, HCH, NHC):
    e = pl.program_id(0)

    @pl.when(e == 0)
    def _():
        x = x_ref[...]
        logits = lax.dot_general(x, wr_ref[...], (((1,), (1,)), ((), ())),
                                 preferred_element_type=jnp.float32)
        logits = logits + br_ref[...]
        m = jnp.max(logits, axis=1, keepdims=True)
        p = jnp.exp(logits - m)
        p = p / jnp.sum(p, axis=1, keepdims=True)

        ie = lax.broadcasted_iota(jnp.int32, (T, E), 1)
        m1 = jnp.max(p, axis=1, keepdims=True)
        am1 = jnp.min(jnp.where(p == m1, ie, E), axis=1, keepdims=True)
        pm = jnp.where(ie == am1, -1.0, p)
        m2 = jnp.max(pm, axis=1, keepdims=True)
        am2 = jnp.min(jnp.where(pm == m2, ie, E), axis=1, keepdims=True)

        oh1 = (ie == am1).astype(jnp.float32)
        oh2 = (ie == am2).astype(jnp.float32)
        both = oh1 + oh2

        it = lax.broadcasted_iota(jnp.int32, (T, T), 0)
        jt = lax.broadcasted_iota(jnp.int32, (T, T), 1)
        Ltri = (jt < it).astype(jnp.float32)
        cnt = lax.dot_general(Ltri, both, (((1,), (0,)), ((), ())),
                              preferred_element_type=jnp.float32, precision=_HIGH)
        r0 = jnp.sum(oh1 * cnt, axis=1, keepdims=True)
        r1 = jnp.sum(oh2 * cnt, axis=1, keepdims=True)

        ne = jnp.sum(both, axis=0, keepdims=True)
        nblk = jnp.ceil(ne * (1.0 / SUB)).astype(jnp.int32)
        for ee in range(E):
            nblk_s[0, ee] = nblk[0, ee]

        f0 = am1 * CAP + (r0 + 0.5).astype(jnp.int32)
        f1 = am2 * CAP + (r1 + 0.5).astype(jnp.int32)
        fcap = lax.broadcasted_iota(jnp.int32, (T, CAP), 1)
        for ee in range(E):
            fi = fcap + ee * CAP
            M0e = (fi == f0).astype(jnp.float32)
            M1e = (fi == f1).astype(jnp.float32)
            mall_s[ee] = M0e + M1e
            mw_s[ee] = M0e * m1 + M1e * m2

        out_ref[...] = jnp.zeros_like(out_ref)

    nblk = nblk_s[0, e]
    xg_ref[...] = lax.dot_general(mall_s[e], x_ref[...],
                                  (((0,), (0,)), ((), ())),
                                  preferred_element_type=jnp.float32)

    for sub in range(CAP // SUB):
        @pl.when(sub < nblk)
        def _():
            xs = xg_ref[sub * SUB:(sub + 1) * SUB, :]
            contrib = None
            for c in range(NHC):
                upr = (up0_ref, up1_ref)[c // (NHC // 2)]
                dnr = (dn0_ref, dn1_ref)[c // (NHC // 2)]
                ci = c % (NHC // 2)
                up_c = upr[0, ci * HCH:(ci + 1) * HCH, :]
                dn_c = dnr[0, :, ci * HCH:(ci + 1) * HCH]
                hp = lax.dot_general(xs, up_c, (((1,), (1,)), ((), ())),
                                     preferred_element_type=jnp.float32)
                g = hp * (0.5 * (1.0 + lax.erf(hp * 0.7071067811865476)))
                d = lax.dot_general(g, dn_c, (((1,), (1,)), ((), ())),
                                    preferred_element_type=jnp.float32)
                contrib = d if contrib is None else contrib + d
            mws = mw_s[e][:, sub * SUB:(sub + 1) * SUB]
            out_ref[...] += lax.dot_general(mws, contrib, (((1,), (0,)), ((), ())),
                                            preferred_element_type=jnp.float32)


def kernel(x, Wr, br, expert_up, expert_down):
    Bsz, Ssz, D = x.shape
    E, H = expert_up.shape[0], expert_up.shape[1]
    T = Bsz * Ssz
    CAP = T
    SUB = 128
    HCH = 768
    NHC = H // HCH
    xf = x.reshape(T, D)

    body = functools.partial(_moe_kernel, T=T, E=E, CAP=CAP, SUB=SUB,
                             HCH=HCH, NHC=NHC)
    out = pl.pallas_call(
        body,
        grid=(E,),
        in_specs=[
            pl.BlockSpec((T, D), lambda e: (0, 0)),
            pl.BlockSpec((E, D), lambda e: (0, 0)),
            pl.BlockSpec((1, E), lambda e: (0, 0)),
            pl.BlockSpec((1, H // 2, D), lambda e: (e, 0, 0)),
            pl.BlockSpec((1, H // 2, D), lambda e: (e, 1, 0)),
            pl.BlockSpec((1, D, H // 2), lambda e: (e, 0, 0)),
            pl.BlockSpec((1, D, H // 2), lambda e: (e, 0, 1)),
        ],
        out_specs=pl.BlockSpec((T, D), lambda e: (0, 0)),
        scratch_shapes=[
            pltpu.VMEM((E, T, CAP), jnp.float32),
            pltpu.VMEM((E, T, CAP), jnp.float32),
            pltpu.VMEM((CAP, D), jnp.float32),
            pltpu.SMEM((1, E), jnp.int32),
        ],
        out_shape=jax.ShapeDtypeStruct((T, D), jnp.float32),
    )(xf, Wr, br.reshape(1, E), expert_up, expert_up, expert_down, expert_down)
    return out.reshape(Bsz, Ssz, D)
